# baseline (speedup 1.0000x reference)
import functools

import jax
import jax.numpy as jnp
from jax.experimental import pallas as pl
from jax.experimental.pallas import tpu as pltpu

_B, _C, _N = 8, 64, 100000
_TN = 12800
_NB = -(-_N // _TN)
_EPS = 1e-5


def _stats_kernel(x_ref, w_ref, g_ref, b_ref, scale_ref, shift_ref,
                  s_acc, q_acc):
    bi = pl.program_id(0)
    ni = pl.program_id(1)

    @pl.when((bi == 0) & (ni == 0))
    def _init():
        s_acc[...] = jnp.zeros_like(s_acc)
        q_acc[...] = jnp.zeros_like(q_acc)

    z = jnp.dot(w_ref[...], x_ref[0], preferred_element_type=jnp.float32)

    @pl.when(ni < _NB - 1)
    def _stats_full():
        s_acc[...] += jnp.sum(z, axis=1, keepdims=True)
        q_acc[...] += jnp.sum(z * z, axis=1, keepdims=True)

    @pl.when(ni == _NB - 1)
    def _stats_tail():
        col = jax.lax.broadcasted_iota(jnp.int32, (_C, _TN), 1)
        zm = jnp.where(col < (_N - ni * _TN), z, 0.0)
        s_acc[...] += jnp.sum(zm, axis=1, keepdims=True)
        q_acc[...] += jnp.sum(zm * zm, axis=1, keepdims=True)

    @pl.when((bi == _B - 1) & (ni == _NB - 1))
    def _finish():
        cnt = jnp.float32(_B * _N)
        mean = s_acc[...] / cnt
        var = q_acc[...] / cnt - mean * mean
        inv = g_ref[...] * jax.lax.rsqrt(var + _EPS)
        scale_ref[...] = inv
        shift_ref[...] = b_ref[...] - mean * inv


def _apply_kernel(x_ref, w_ref, scale_ref, shift_ref, o_ref):
    z = jnp.dot(w_ref[...], x_ref[0], preferred_element_type=jnp.float32)
    o_ref[0] = jnp.maximum(z * scale_ref[...] + shift_ref[...], 0.0)


@functools.partial(jax.jit, static_argnames=())
def _run(x, W, gamma, beta):
    g2 = gamma.reshape(_C, 1)
    b2 = beta.reshape(_C, 1)

    scale, shift = pl.pallas_call(
        _stats_kernel,
        grid=(_B, _NB),
        in_specs=[
            pl.BlockSpec((1, _C, _TN), lambda b, n: (b, 0, n)),
            pl.BlockSpec((_C, _C), lambda b, n: (0, 0)),
            pl.BlockSpec((_C, 1), lambda b, n: (0, 0)),
            pl.BlockSpec((_C, 1), lambda b, n: (0, 0)),
        ],
        out_specs=[
            pl.BlockSpec((_C, 1), lambda b, n: (0, 0)),
            pl.BlockSpec((_C, 1), lambda b, n: (0, 0)),
        ],
        out_shape=[
            jax.ShapeDtypeStruct((_C, 1), jnp.float32),
            jax.ShapeDtypeStruct((_C, 1), jnp.float32),
        ],
        scratch_shapes=[
            pltpu.VMEM((_C, 1), jnp.float32),
            pltpu.VMEM((_C, 1), jnp.float32),
        ],
    )(x, W, g2, b2)

    y = pl.pallas_call(
        _apply_kernel,
        grid=(_B, _NB),
        in_specs=[
            pl.BlockSpec((1, _C, _TN), lambda b, n: (b, 0, n)),
            pl.BlockSpec((_C, _C), lambda b, n: (0, 0)),
            pl.BlockSpec((_C, 1), lambda b, n: (0, 0)),
            pl.BlockSpec((_C, 1), lambda b, n: (0, 0)),
        ],
        out_specs=pl.BlockSpec((1, _C, _TN), lambda b, n: (b, 0, n)),
        out_shape=jax.ShapeDtypeStruct((_B, _C, _N), jnp.float32),
    )(x, W, scale, shift)

    return y


def kernel(p, x, W, gamma, beta):
    return (p, _run(x, W, gamma, beta))


# speedup vs baseline: 1.1448x; 1.1448x over previous
import functools

import jax
import jax.numpy as jnp
from jax.experimental import pallas as pl
from jax.experimental.pallas import tpu as pltpu

_B, _C, _N = 8, 64, 100000
_TN = 50048
_NB = -(-_N // _TN)
_EPS = 1e-5


def _stats_kernel(x_ref, w_ref, g_ref, b_ref, scale_ref, shift_ref,
                  s_acc, q_acc):
    bi = pl.program_id(0)
    ni = pl.program_id(1)

    @pl.when((bi == 0) & (ni == 0))
    def _init():
        s_acc[...] = jnp.zeros_like(s_acc)
        q_acc[...] = jnp.zeros_like(q_acc)

    z = jnp.dot(w_ref[...], x_ref[0], preferred_element_type=jnp.float32)

    @pl.when(ni < _NB - 1)
    def _stats_full():
        s_acc[...] += jnp.sum(z, axis=1, keepdims=True)
        q_acc[...] += jnp.sum(z * z, axis=1, keepdims=True)

    @pl.when(ni == _NB - 1)
    def _stats_tail():
        col = jax.lax.broadcasted_iota(jnp.int32, (_C, _TN), 1)
        zm = jnp.where(col < (_N - ni * _TN), z, 0.0)
        s_acc[...] += jnp.sum(zm, axis=1, keepdims=True)
        q_acc[...] += jnp.sum(zm * zm, axis=1, keepdims=True)

    @pl.when((bi == _B - 1) & (ni == _NB - 1))
    def _finish():
        cnt = jnp.float32(_B * _N)
        mean = s_acc[...] / cnt
        var = q_acc[...] / cnt - mean * mean
        inv = g_ref[...] * jax.lax.rsqrt(var + _EPS)
        scale_ref[...] = inv
        shift_ref[...] = b_ref[...] - mean * inv


def _apply_kernel(x_ref, w_ref, scale_ref, shift_ref, o_ref):
    z = jnp.dot(w_ref[...], x_ref[0], preferred_element_type=jnp.float32)
    o_ref[0] = jnp.maximum(z * scale_ref[...] + shift_ref[...], 0.0)


@functools.partial(jax.jit, static_argnames=())
def _run(x, W, gamma, beta):
    g2 = gamma.reshape(_C, 1)
    b2 = beta.reshape(_C, 1)

    scale, shift = pl.pallas_call(
        _stats_kernel,
        grid=(_B, _NB),
        in_specs=[
            pl.BlockSpec((1, _C, _TN), lambda b, n: (b, 0, n)),
            pl.BlockSpec((_C, _C), lambda b, n: (0, 0)),
            pl.BlockSpec((_C, 1), lambda b, n: (0, 0)),
            pl.BlockSpec((_C, 1), lambda b, n: (0, 0)),
        ],
        out_specs=[
            pl.BlockSpec((_C, 1), lambda b, n: (0, 0)),
            pl.BlockSpec((_C, 1), lambda b, n: (0, 0)),
        ],
        out_shape=[
            jax.ShapeDtypeStruct((_C, 1), jnp.float32),
            jax.ShapeDtypeStruct((_C, 1), jnp.float32),
        ],
        scratch_shapes=[
            pltpu.VMEM((_C, 1), jnp.float32),
            pltpu.VMEM((_C, 1), jnp.float32),
        ],
    )(x, W, g2, b2)

    y = pl.pallas_call(
        _apply_kernel,
        grid=(_B, _NB),
        in_specs=[
            pl.BlockSpec((1, _C, _TN), lambda b, n: (b, 0, n)),
            pl.BlockSpec((_C, _C), lambda b, n: (0, 0)),
            pl.BlockSpec((_C, 1), lambda b, n: (0, 0)),
            pl.BlockSpec((_C, 1), lambda b, n: (0, 0)),
        ],
        out_specs=pl.BlockSpec((1, _C, _TN), lambda b, n: (b, 0, n)),
        out_shape=jax.ShapeDtypeStruct((_B, _C, _N), jnp.float32),
    )(x, W, scale, shift)

    return y


def kernel(p, x, W, gamma, beta):
    return (p, _run(x, W, gamma, beta))


# speedup vs baseline: 1.1945x; 1.0435x over previous
import functools

import jax
import jax.numpy as jnp
from jax.experimental import pallas as pl
from jax.experimental.pallas import tpu as pltpu

_B, _C, _N = 8, 64, 100000
_TN = 25600
_NB = -(-_N // _TN)
_TOT = _B * _NB
_PARK = 8
_REF = _TOT - _PARK
_EPS = 1e-5

_PIN_B, _PIN_N = (_REF - 1) // _NB, (_REF - 1) % _NB


def _fused_kernel(x_ref, w_ref, g_ref, b_ref, o_ref,
                  zs, s_acc, q_acc, w2_s, scale_s, shift_s):
    ph = pl.program_id(0)
    bi = pl.program_id(1)
    ni = pl.program_id(2)
    idx = bi * _NB + ni

    @pl.when((ph == 0) & (idx == 0))
    def _init():
        s_acc[...] = jnp.zeros_like(s_acc)
        q_acc[...] = jnp.zeros_like(q_acc)

    @pl.when(ph == 0)
    def _sweep():
        z = jnp.dot(w_ref[...], x_ref[0], preferred_element_type=jnp.float32)

        @pl.when(ni < _NB - 1)
        def _stats_full():
            s_acc[...] += jnp.sum(z, axis=1, keepdims=True)
            q_acc[...] += jnp.sum(z * z, axis=1, keepdims=True)

        @pl.when(ni == _NB - 1)
        def _stats_tail():
            col = jax.lax.broadcasted_iota(jnp.int32, (_C, _TN), 1)
            zm = jnp.where(col < (_N - ni * _TN), z, 0.0)
            s_acc[...] += jnp.sum(zm, axis=1, keepdims=True)
            q_acc[...] += jnp.sum(zm * zm, axis=1, keepdims=True)

        @pl.when(idx >= _REF)
        def _park():
            zs[jnp.maximum(idx - _REF, 0)] = z.astype(jnp.bfloat16)

    @pl.when((ph == 0) & (idx == _TOT - 1))
    def _finish_stats():
        cnt = jnp.float32(_B * _N)
        mean = s_acc[...] / cnt
        var = q_acc[...] / cnt - mean * mean
        inv = g_ref[...] * jax.lax.rsqrt(var + _EPS)
        scale_s[...] = inv
        shift_s[...] = b_ref[...] - mean * inv
        w2_s[...] = w_ref[...] * inv

    @pl.when((ph == 1) & (idx < _REF))
    def _recompute():
        z = jnp.dot(w2_s[...], x_ref[0], preferred_element_type=jnp.float32)
        o_ref[0] = jnp.maximum(z + shift_s[...], 0.0)

    @pl.when((ph == 1) & (idx >= _REF))
    def _replay():
        z = zs[jnp.maximum(idx - _REF, 0)].astype(jnp.float32)
        o_ref[0] = jnp.maximum(z * scale_s[...] + shift_s[...], 0.0)


def _x_index_map(p, b, n):
    idx = b * _NB + n
    pinned = (p == 1) & (idx >= _REF)
    return (jnp.where(pinned, _PIN_B, b), 0, jnp.where(pinned, _PIN_N, n))


def _out_index_map(p, b, n):
    return (jnp.where(p == 0, 0, b), 0, jnp.where(p == 0, 0, n))


@functools.partial(jax.jit, static_argnames=())
def _run(x, W, gamma, beta):
    g2 = gamma.reshape(_C, 1)
    b2 = beta.reshape(_C, 1)

    y = pl.pallas_call(
        _fused_kernel,
        grid=(2, _B, _NB),
        in_specs=[
            pl.BlockSpec((1, _C, _TN), _x_index_map),
            pl.BlockSpec((_C, _C), lambda p, b, n: (0, 0)),
            pl.BlockSpec((_C, 1), lambda p, b, n: (0, 0)),
            pl.BlockSpec((_C, 1), lambda p, b, n: (0, 0)),
        ],
        out_specs=pl.BlockSpec((1, _C, _TN), _out_index_map),
        out_shape=jax.ShapeDtypeStruct((_B, _C, _N), jnp.float32),
        scratch_shapes=[
            pltpu.VMEM((_PARK, _C, _TN), jnp.bfloat16),
            pltpu.VMEM((_C, 1), jnp.float32),
            pltpu.VMEM((_C, 1), jnp.float32),
            pltpu.VMEM((_C, _C), jnp.float32),
            pltpu.VMEM((_C, 1), jnp.float32),
            pltpu.VMEM((_C, 1), jnp.float32),
        ],
        compiler_params=pltpu.CompilerParams(
            vmem_limit_bytes=64 * 1024 * 1024,
        ),
    )(x, W, g2, b2)

    return y


def kernel(p, x, W, gamma, beta):
    return (p, _run(x, W, gamma, beta))
